# Initial kernel scaffold; baseline (speedup 1.0000x reference)
import jax
import jax.numpy as jnp
from jax.experimental import pallas as pl


def kernel(x, edge_index, batch, graph_features, W1, b1, W2, b2, fcW, fcb):
    raise NotImplementedError("write your pallas kernel here")


# speedup vs baseline: 24.5256x; 24.5256x over previous
import functools

import jax
import jax.numpy as jnp
from jax import lax
from jax.experimental import pallas as pl
from jax.experimental.pallas import tpu as pltpu
from jax.experimental.pallas import tpu_sc as plsc

N = 10000
E = 320000
D_FEAT = 128
H1 = 16
H2 = 32
G = 64
GF = 16
NCLS = 10

NC = 2
NS = 16
NW = NC * NS
CH = 128
EPW = 10240
KCH = EPW // CH
EPAD = NW * EPW
NPAD = 10240
RPT = NPAD // NS

def _mesh():
    return plsc.VectorSubcoreMesh(
        core_axis_name="c", subcore_axis_name="s",
        num_cores=NC, num_subcores=NS)


@functools.cache
def _make_deg_kernel():
    @functools.partial(
        pl.kernel,
        out_type=jax.ShapeDtypeStruct((NC, NPAD), jnp.float32),
        mesh=_mesh(),
        compiler_params=pltpu.CompilerParams(use_tc_tiling_on_sc=False),
        scratch_types=[
            pltpu.VMEM((KCH, CH), jnp.int32),
            pltpu.VMEM((CH,), jnp.float32),
            pltpu.VMEM((RPT,), jnp.float32),
            pltpu.VMEM_SHARED((NPAD,), jnp.float32),
        ])
    def _deg_kernel(dst_hbm, out_hbm, dst_v, ones_v, zbuf, acc_sh):
        cid = lax.axis_index("c")
        sid = lax.axis_index("s")
        wid = sid * NC + cid

        one16 = jnp.ones((16,), jnp.float32)
        zero16 = jnp.zeros((16,), jnp.float32)

        def fill(i, _):
            ones_v[pl.ds(i * 16, 16)] = one16
            return 0
        lax.fori_loop(0, CH // 16, fill, 0)

        def zfill(i, _):
            zbuf[pl.ds(i * 16, 16)] = zero16
            return 0
        lax.fori_loop(0, RPT // 16, zfill, 0)
        pltpu.sync_copy(zbuf, acc_sh.at[pl.ds(sid * RPT, RPT)])
        plsc.subcore_barrier()

        pltpu.sync_copy(dst_hbm.at[pl.ds(wid * KCH, KCH)], dst_v)

        def body(j, _):
            pltpu.sync_copy(ones_v, acc_sh.at[dst_v.at[j]], add=True)
            return 0
        lax.fori_loop(0, KCH, body, 0)
        plsc.subcore_barrier()

        pltpu.sync_copy(acc_sh.at[pl.ds(sid * RPT, RPT)],
                        out_hbm.at[cid].at[pl.ds(sid * RPT, RPT)])
    return _deg_kernel


@functools.cache
def _make_scatter_kernel(F):
    @functools.partial(
        pl.kernel,
        out_type=jax.ShapeDtypeStruct((NC, NPAD, F), jnp.float32),
        mesh=_mesh(),
        compiler_params=pltpu.CompilerParams(use_tc_tiling_on_sc=False),
        scratch_types=[
            pltpu.VMEM((KCH, CH), jnp.int32),
            pltpu.VMEM((KCH, CH), jnp.int32),
            pltpu.VMEM((CH, F), jnp.float32),
            pltpu.VMEM((RPT, F), jnp.float32),
            pltpu.VMEM_SHARED((NPAD, F), jnp.float32),
            pltpu.SemaphoreType.DMA,
        ])
    def k(table_hbm, src_hbm, dst_hbm, out_hbm,
          src_v, dst_v, rows_v, zbuf, acc_sh, sem):
        cid = lax.axis_index("c")
        sid = lax.axis_index("s")
        wid = sid * NC + cid

        zero16 = jnp.zeros((16,), jnp.float32)

        def zfill(i, _):
            for j in range(F // 16):
                zbuf[i, pl.ds(j * 16, 16)] = zero16
            return 0
        lax.fori_loop(0, RPT, zfill, 0)
        pltpu.sync_copy(zbuf, acc_sh.at[pl.ds(sid * RPT, RPT)])
        plsc.subcore_barrier()

        pltpu.sync_copy(src_hbm.at[pl.ds(wid * KCH, KCH)], src_v)
        pltpu.sync_copy(dst_hbm.at[pl.ds(wid * KCH, KCH)], dst_v)

        def body(j, _):
            pltpu.async_copy(table_hbm.at[src_v.at[j]], rows_v, sem).wait()
            pltpu.sync_copy(rows_v, acc_sh.at[dst_v.at[j]], add=True)
            return 0
        lax.fori_loop(0, KCH, body, 0)
        plsc.subcore_barrier()

        pltpu.sync_copy(acc_sh.at[pl.ds(sid * RPT, RPT)],
                        out_hbm.at[cid].at[pl.ds(sid * RPT, RPT)])
    return k


def _k1_body(x_ref, w1_ref, degp_ref, dinv_ref, hs1_ref):
    degp = degp_ref[...]
    deg = degp[:, 0:1] + degp[:, 1:2] + 1.0
    dinv = lax.rsqrt(deg)
    dinv_ref[...] = dinv
    h = jnp.dot(x_ref[...], w1_ref[...], preferred_element_type=jnp.float32)
    hs1_ref[...] = h * dinv[:N]


def _k3_body(accp_ref, hs1_ref, dinv_ref, b1_ref, w2_ref, hs2_ref):
    acc = accp_ref[0, :N, :] + accp_ref[1, :N, :]
    dinv = dinv_ref[...][:N]
    t = jax.nn.relu(dinv * (acc + hs1_ref[...]) + b1_ref[...])
    hs2_ref[...] = jnp.dot(t, w2_ref[...],
                           preferred_element_type=jnp.float32) * dinv


def _k5_body(accp_ref, hs2_ref, dinv_ref, b2_ref, batch_ref, gf_ref,
             fcWh_ref, fcWg_ref, fcb_ref, out_ref):
    acc = accp_ref[0, :N, :] + accp_ref[1, :N, :]
    dinv = dinv_ref[...][:N]
    h2 = jax.nn.relu(dinv * (acc + hs2_ref[...]) + b2_ref[...])
    gid = lax.broadcasted_iota(jnp.int32, (G, N), 0)
    onehot = (gid == batch_ref[...]).astype(jnp.float32)
    counts = jnp.sum(onehot, axis=1, keepdims=True)
    sums = jnp.dot(onehot, h2, preferred_element_type=jnp.float32)
    pooled = sums / jnp.maximum(counts, 1.0)
    z = (jnp.dot(pooled, fcWh_ref[...], preferred_element_type=jnp.float32)
         + jnp.dot(gf_ref[...], fcWg_ref[...],
                   preferred_element_type=jnp.float32)
         + fcb_ref[...])
    m = jnp.max(z, axis=1, keepdims=True)
    lse = m + jnp.log(jnp.sum(jnp.exp(z - m), axis=1, keepdims=True))
    out_ref[...] = z - lse


def kernel(x, edge_index, batch, graph_features, W1, b1, W2, b2, fcW, fcb):
    src = edge_index[0].astype(jnp.int32)
    dst = edge_index[1].astype(jnp.int32)
    npad = EPAD - E
    srcp = jnp.concatenate([src, jnp.zeros((npad,), jnp.int32)])
    dstp = jnp.concatenate([dst, jnp.full((npad,), N, jnp.int32)])
    srcm = srcp.reshape(NW * KCH, CH)
    dstm = dstp.reshape(NW * KCH, CH)

    degp = _make_deg_kernel()(dstm)
    degp_t = degp.T

    dinv, hs1 = pl.pallas_call(
        _k1_body,
        out_shape=[jax.ShapeDtypeStruct((NPAD, 1), jnp.float32),
                   jax.ShapeDtypeStruct((N, H1), jnp.float32)],
    )(x, W1, degp_t)

    acc1 = _make_scatter_kernel(H1)(hs1, srcm, dstm)

    hs2 = pl.pallas_call(
        _k3_body,
        out_shape=jax.ShapeDtypeStruct((N, H2), jnp.float32),
    )(acc1, hs1, dinv, b1.reshape(1, H1), W2)

    acc2 = _make_scatter_kernel(H2)(hs2, srcm, dstm)

    out = pl.pallas_call(
        _k5_body,
        out_shape=jax.ShapeDtypeStruct((G, NCLS), jnp.float32),
    )(acc2, hs2, dinv, b2.reshape(1, H2), batch.astype(jnp.int32).reshape(1, N),
      graph_features, fcW[:H2], fcW[H2:], fcb.reshape(1, NCLS))

    return out


# speedup vs baseline: 30.3471x; 1.2374x over previous
import functools

import jax
import jax.numpy as jnp
from jax import lax
from jax.experimental import pallas as pl
from jax.experimental.pallas import tpu as pltpu
from jax.experimental.pallas import tpu_sc as plsc

N = 10000
E = 320000
D_FEAT = 128
H1 = 16
H2 = 32
G = 64
GF = 16
NCLS = 10

NC = 2
NS = 16
NW = NC * NS
CH = 128
EPW = 10240
KCH = EPW // CH
EPAD = NW * EPW
NPAD = 10240
RPT = NPAD // NS
NB = 4

def _mesh():
    return plsc.VectorSubcoreMesh(
        core_axis_name="c", subcore_axis_name="s",
        num_cores=NC, num_subcores=NS)


@functools.cache
def _make_deg_kernel():
    @functools.partial(
        pl.kernel,
        out_type=jax.ShapeDtypeStruct((NC, NPAD), jnp.float32),
        mesh=_mesh(),
        compiler_params=pltpu.CompilerParams(use_tc_tiling_on_sc=False),
        scratch_types=[
            pltpu.VMEM((KCH, CH), jnp.int32),
            pltpu.VMEM((CH,), jnp.float32),
            pltpu.VMEM((RPT,), jnp.float32),
            pltpu.VMEM_SHARED((NPAD,), jnp.float32),
        ])
    def _deg_kernel(dst_hbm, out_hbm, dst_v, ones_v, zbuf, acc_sh):
        cid = lax.axis_index("c")
        sid = lax.axis_index("s")
        wid = sid * NC + cid

        one16 = jnp.ones((16,), jnp.float32)
        zero16 = jnp.zeros((16,), jnp.float32)

        def fill(i, _):
            ones_v[pl.ds(i * 16, 16)] = one16
            return 0
        lax.fori_loop(0, CH // 16, fill, 0)

        def zfill(i, _):
            zbuf[pl.ds(i * 16, 16)] = zero16
            return 0
        lax.fori_loop(0, RPT // 16, zfill, 0)
        pltpu.sync_copy(zbuf, acc_sh.at[pl.ds(sid * RPT, RPT)])
        plsc.subcore_barrier()

        pltpu.sync_copy(dst_hbm.at[pl.ds(wid * KCH, KCH)], dst_v)

        def body(j, _):
            pltpu.sync_copy(ones_v, acc_sh.at[dst_v.at[j]], add=True)
            return 0
        lax.fori_loop(0, KCH, body, 0)
        plsc.subcore_barrier()

        pltpu.sync_copy(acc_sh.at[pl.ds(sid * RPT, RPT)],
                        out_hbm.at[cid].at[pl.ds(sid * RPT, RPT)])
    return _deg_kernel


@functools.cache
def _make_scatter_kernel(F):
    @functools.partial(
        pl.kernel,
        out_type=jax.ShapeDtypeStruct((NC, NPAD, F), jnp.float32),
        mesh=_mesh(),
        compiler_params=pltpu.CompilerParams(use_tc_tiling_on_sc=False),
        scratch_types=[
            pltpu.VMEM((KCH, CH), jnp.int32),
            pltpu.VMEM((KCH, CH), jnp.int32),
            pltpu.VMEM((2 * NB, CH, F), jnp.float32),
            pltpu.VMEM((RPT, F), jnp.float32),
            pltpu.VMEM_SHARED((NPAD, F), jnp.float32),
            pltpu.SemaphoreType.DMA,
            pltpu.SemaphoreType.DMA,
        ])
    def k(table_hbm, src_hbm, dst_hbm, out_hbm,
          src_v, dst_v, rows_v, zbuf, acc_sh, sem_g, sem_s):
        cid = lax.axis_index("c")
        sid = lax.axis_index("s")
        wid = sid * NC + cid

        zero16 = jnp.zeros((16,), jnp.float32)

        def zfill(i, _):
            for j in range(F // 16):
                zbuf[i, pl.ds(j * 16, 16)] = zero16
            return 0
        lax.fori_loop(0, RPT, zfill, 0)
        pltpu.sync_copy(zbuf, acc_sh.at[pl.ds(sid * RPT, RPT)])
        plsc.subcore_barrier()

        pltpu.sync_copy(src_hbm.at[pl.ds(wid * KCH, KCH)], src_v)
        pltpu.sync_copy(dst_hbm.at[pl.ds(wid * KCH, KCH)], dst_v)

        NG = KCH // NB

        for i in range(NB):
            pltpu.async_copy(table_hbm.at[src_v.at[i]], rows_v.at[i], sem_g)

        def body(g, _):
            p = (g % 2) * NB
            q = ((g + 1) % 2) * NB
            for i in range(NB):
                pltpu.make_async_copy(
                    table_hbm.at[src_v.at[i]], rows_v.at[p + i], sem_g).wait()
            @pl.when(g > 0)
            def _():
                for i in range(NB):
                    pltpu.make_async_copy(
                        rows_v.at[q + i], acc_sh.at[dst_v.at[i]], sem_s).wait()
            @pl.when(g + 1 < NG)
            def _():
                for i in range(NB):
                    pltpu.async_copy(
                        table_hbm.at[src_v.at[(g + 1) * NB + i]],
                        rows_v.at[q + i], sem_g)
            for i in range(NB):
                pltpu.async_copy(rows_v.at[p + i],
                                 acc_sh.at[dst_v.at[g * NB + i]], sem_s,
                                 add=True)
            return 0
        lax.fori_loop(0, NG, body, 0)
        for i in range(NB):
            pltpu.make_async_copy(
                rows_v.at[i], acc_sh.at[dst_v.at[i]], sem_s).wait()
        plsc.subcore_barrier()

        pltpu.sync_copy(acc_sh.at[pl.ds(sid * RPT, RPT)],
                        out_hbm.at[cid].at[pl.ds(sid * RPT, RPT)])
    return k


def _k1_body(x_ref, w1_ref, degp_ref, dinv_ref, hs1_ref):
    degp = degp_ref[...]
    deg = degp[:, 0:1] + degp[:, 1:2] + 1.0
    dinv = lax.rsqrt(deg)
    dinv_ref[...] = dinv
    h = jnp.dot(x_ref[...], w1_ref[...], preferred_element_type=jnp.float32)
    hs1_ref[...] = h * dinv[:N]


def _k3_body(accp_ref, hs1_ref, dinv_ref, b1_ref, w2_ref, hs2_ref):
    acc = accp_ref[0, :N, :] + accp_ref[1, :N, :]
    dinv = dinv_ref[...][:N]
    t = jax.nn.relu(dinv * (acc + hs1_ref[...]) + b1_ref[...])
    hs2_ref[...] = jnp.dot(t, w2_ref[...],
                           preferred_element_type=jnp.float32) * dinv


def _k5_body(accp_ref, hs2_ref, dinv_ref, b2_ref, batch_ref, gf_ref,
             fcWh_ref, fcWg_ref, fcb_ref, out_ref):
    acc = accp_ref[0, :N, :] + accp_ref[1, :N, :]
    dinv = dinv_ref[...][:N]
    h2 = jax.nn.relu(dinv * (acc + hs2_ref[...]) + b2_ref[...])
    gid = lax.broadcasted_iota(jnp.int32, (G, N), 0)
    onehot = (gid == batch_ref[...]).astype(jnp.float32)
    counts = jnp.sum(onehot, axis=1, keepdims=True)
    sums = jnp.dot(onehot, h2, preferred_element_type=jnp.float32)
    pooled = sums / jnp.maximum(counts, 1.0)
    z = (jnp.dot(pooled, fcWh_ref[...], preferred_element_type=jnp.float32)
         + jnp.dot(gf_ref[...], fcWg_ref[...],
                   preferred_element_type=jnp.float32)
         + fcb_ref[...])
    m = jnp.max(z, axis=1, keepdims=True)
    lse = m + jnp.log(jnp.sum(jnp.exp(z - m), axis=1, keepdims=True))
    out_ref[...] = z - lse


def kernel(x, edge_index, batch, graph_features, W1, b1, W2, b2, fcW, fcb):
    src = edge_index[0].astype(jnp.int32)
    dst = edge_index[1].astype(jnp.int32)
    npad = EPAD - E
    srcp = jnp.concatenate([src, jnp.zeros((npad,), jnp.int32)])
    dstp = jnp.concatenate([dst, jnp.full((npad,), N, jnp.int32)])
    srcm = srcp.reshape(NW * KCH, CH)
    dstm = dstp.reshape(NW * KCH, CH)

    degp = _make_deg_kernel()(dstm)
    degp_t = degp.T

    dinv, hs1 = pl.pallas_call(
        _k1_body,
        out_shape=[jax.ShapeDtypeStruct((NPAD, 1), jnp.float32),
                   jax.ShapeDtypeStruct((N, H1), jnp.float32)],
    )(x, W1, degp_t)

    acc1 = _make_scatter_kernel(H1)(hs1, srcm, dstm)

    hs2 = pl.pallas_call(
        _k3_body,
        out_shape=jax.ShapeDtypeStruct((N, H2), jnp.float32),
    )(acc1, hs1, dinv, b1.reshape(1, H1), W2)

    acc2 = _make_scatter_kernel(H2)(hs2, srcm, dstm)

    out = pl.pallas_call(
        _k5_body,
        out_shape=jax.ShapeDtypeStruct((G, NCLS), jnp.float32),
    )(acc2, hs2, dinv, b2.reshape(1, H2), batch.astype(jnp.int32).reshape(1, N),
      graph_features, fcW[:H2], fcW[H2:], fcb.reshape(1, NCLS))

    return out


# speedup vs baseline: 31.0649x; 1.0237x over previous
import functools

import jax
import jax.numpy as jnp
from jax import lax
from jax.experimental import pallas as pl
from jax.experimental.pallas import tpu as pltpu
from jax.experimental.pallas import tpu_sc as plsc

N = 10000
E = 320000
D_FEAT = 128
H1 = 16
H2 = 32
G = 64
GF = 16
NCLS = 10

NC = 2
NS = 16
NW = NC * NS
CH = 128
EPW = 10240
KCH = EPW // CH
EPAD = NW * EPW
NPAD = 10240
RPT = NPAD // NS
NB = 8

def _mesh():
    return plsc.VectorSubcoreMesh(
        core_axis_name="c", subcore_axis_name="s",
        num_cores=NC, num_subcores=NS)


@functools.cache
def _make_deg_kernel():
    @functools.partial(
        pl.kernel,
        out_type=jax.ShapeDtypeStruct((NC, NPAD), jnp.float32),
        mesh=_mesh(),
        compiler_params=pltpu.CompilerParams(use_tc_tiling_on_sc=False),
        scratch_types=[
            pltpu.VMEM((KCH, CH), jnp.int32),
            pltpu.VMEM((CH,), jnp.float32),
            pltpu.VMEM((RPT,), jnp.float32),
            pltpu.VMEM_SHARED((NPAD,), jnp.float32),
        ])
    def _deg_kernel(dst_hbm, out_hbm, dst_v, ones_v, zbuf, acc_sh):
        cid = lax.axis_index("c")
        sid = lax.axis_index("s")
        wid = sid * NC + cid

        one16 = jnp.ones((16,), jnp.float32)
        zero16 = jnp.zeros((16,), jnp.float32)

        def fill(i, _):
            ones_v[pl.ds(i * 16, 16)] = one16
            return 0
        lax.fori_loop(0, CH // 16, fill, 0)

        def zfill(i, _):
            zbuf[pl.ds(i * 16, 16)] = zero16
            return 0
        lax.fori_loop(0, RPT // 16, zfill, 0)
        pltpu.sync_copy(zbuf, acc_sh.at[pl.ds(sid * RPT, RPT)])
        plsc.subcore_barrier()

        pltpu.sync_copy(dst_hbm.at[pl.ds(wid * KCH, KCH)], dst_v)

        def body(j, _):
            pltpu.sync_copy(ones_v, acc_sh.at[dst_v.at[j]], add=True)
            return 0
        lax.fori_loop(0, KCH, body, 0)
        plsc.subcore_barrier()

        pltpu.sync_copy(acc_sh.at[pl.ds(sid * RPT, RPT)],
                        out_hbm.at[cid].at[pl.ds(sid * RPT, RPT)])
    return _deg_kernel


@functools.cache
def _make_scatter_kernel(F):
    @functools.partial(
        pl.kernel,
        out_type=jax.ShapeDtypeStruct((NC, NPAD, F), jnp.float32),
        mesh=_mesh(),
        compiler_params=pltpu.CompilerParams(use_tc_tiling_on_sc=False),
        scratch_types=[
            pltpu.VMEM((KCH, CH), jnp.int32),
            pltpu.VMEM((KCH, CH), jnp.int32),
            pltpu.VMEM((2 * NB, CH, F), jnp.float32),
            pltpu.VMEM((RPT, F), jnp.float32),
            pltpu.VMEM_SHARED((NPAD, F), jnp.float32),
            pltpu.SemaphoreType.DMA,
            pltpu.SemaphoreType.DMA,
        ])
    def k(table_hbm, src_hbm, dst_hbm, out_hbm,
          src_v, dst_v, rows_v, zbuf, acc_sh, sem_g, sem_s):
        cid = lax.axis_index("c")
        sid = lax.axis_index("s")
        wid = sid * NC + cid

        zero16 = jnp.zeros((16,), jnp.float32)

        def zfill(i, _):
            for j in range(F // 16):
                zbuf[i, pl.ds(j * 16, 16)] = zero16
            return 0
        lax.fori_loop(0, RPT, zfill, 0)
        pltpu.sync_copy(zbuf, acc_sh.at[pl.ds(sid * RPT, RPT)])
        plsc.subcore_barrier()

        pltpu.sync_copy(src_hbm.at[pl.ds(wid * KCH, KCH)], src_v)
        pltpu.sync_copy(dst_hbm.at[pl.ds(wid * KCH, KCH)], dst_v)

        NG = KCH // NB

        for i in range(NB):
            pltpu.async_copy(table_hbm.at[src_v.at[i]], rows_v.at[i], sem_g)

        def body(g, _):
            p = (g % 2) * NB
            q = ((g + 1) % 2) * NB
            for i in range(NB):
                pltpu.make_async_copy(
                    table_hbm.at[src_v.at[i]], rows_v.at[p + i], sem_g).wait()
            @pl.when(g > 0)
            def _():
                for i in range(NB):
                    pltpu.make_async_copy(
                        rows_v.at[q + i], acc_sh.at[dst_v.at[i]], sem_s).wait()
            @pl.when(g + 1 < NG)
            def _():
                for i in range(NB):
                    pltpu.async_copy(
                        table_hbm.at[src_v.at[(g + 1) * NB + i]],
                        rows_v.at[q + i], sem_g)
            for i in range(NB):
                pltpu.async_copy(rows_v.at[p + i],
                                 acc_sh.at[dst_v.at[g * NB + i]], sem_s,
                                 add=True)
            return 0
        lax.fori_loop(0, NG, body, 0)
        for i in range(NB):
            pltpu.make_async_copy(
                rows_v.at[i], acc_sh.at[dst_v.at[i]], sem_s).wait()
        plsc.subcore_barrier()

        pltpu.sync_copy(acc_sh.at[pl.ds(sid * RPT, RPT)],
                        out_hbm.at[cid].at[pl.ds(sid * RPT, RPT)])
    return k


def _k1_body(x_ref, w1_ref, degp_ref, dinv_ref, hs1_ref):
    degp = degp_ref[...]
    deg = degp[:, 0:1] + degp[:, 1:2] + 1.0
    dinv = lax.rsqrt(deg)
    dinv_ref[...] = dinv
    h = jnp.dot(x_ref[...], w1_ref[...], preferred_element_type=jnp.float32)
    hs1_ref[...] = h * dinv[:N]


def _k3_body(accp_ref, hs1_ref, dinv_ref, b1_ref, w2_ref, hs2_ref):
    acc = accp_ref[0, :N, :] + accp_ref[1, :N, :]
    dinv = dinv_ref[...][:N]
    t = jax.nn.relu(dinv * (acc + hs1_ref[...]) + b1_ref[...])
    hs2_ref[...] = jnp.dot(t, w2_ref[...],
                           preferred_element_type=jnp.float32) * dinv


def _k5_body(accp_ref, hs2_ref, dinv_ref, b2_ref, batch_ref, gf_ref,
             fcWh_ref, fcWg_ref, fcb_ref, out_ref):
    acc = accp_ref[0, :N, :] + accp_ref[1, :N, :]
    dinv = dinv_ref[...][:N]
    h2 = jax.nn.relu(dinv * (acc + hs2_ref[...]) + b2_ref[...])
    gid = lax.broadcasted_iota(jnp.int32, (G, N), 0)
    onehot = (gid == batch_ref[...]).astype(jnp.float32)
    counts = jnp.sum(onehot, axis=1, keepdims=True)
    sums = jnp.dot(onehot, h2, preferred_element_type=jnp.float32)
    pooled = sums / jnp.maximum(counts, 1.0)
    z = (jnp.dot(pooled, fcWh_ref[...], preferred_element_type=jnp.float32)
         + jnp.dot(gf_ref[...], fcWg_ref[...],
                   preferred_element_type=jnp.float32)
         + fcb_ref[...])
    m = jnp.max(z, axis=1, keepdims=True)
    lse = m + jnp.log(jnp.sum(jnp.exp(z - m), axis=1, keepdims=True))
    out_ref[...] = z - lse


def kernel(x, edge_index, batch, graph_features, W1, b1, W2, b2, fcW, fcb):
    src = edge_index[0].astype(jnp.int32)
    dst = edge_index[1].astype(jnp.int32)
    npad = EPAD - E
    srcp = jnp.concatenate([src, jnp.zeros((npad,), jnp.int32)])
    dstp = jnp.concatenate([dst, jnp.full((npad,), N, jnp.int32)])
    srcm = srcp.reshape(NW * KCH, CH)
    dstm = dstp.reshape(NW * KCH, CH)

    degp = _make_deg_kernel()(dstm)
    degp_t = degp.T

    dinv, hs1 = pl.pallas_call(
        _k1_body,
        out_shape=[jax.ShapeDtypeStruct((NPAD, 1), jnp.float32),
                   jax.ShapeDtypeStruct((N, H1), jnp.float32)],
    )(x, W1, degp_t)

    acc1 = _make_scatter_kernel(H1)(hs1, srcm, dstm)

    hs2 = pl.pallas_call(
        _k3_body,
        out_shape=jax.ShapeDtypeStruct((N, H2), jnp.float32),
    )(acc1, hs1, dinv, b1.reshape(1, H1), W2)

    acc2 = _make_scatter_kernel(H2)(hs2, srcm, dstm)

    out = pl.pallas_call(
        _k5_body,
        out_shape=jax.ShapeDtypeStruct((G, NCLS), jnp.float32),
    )(acc2, hs2, dinv, b2.reshape(1, H2), batch.astype(jnp.int32).reshape(1, N),
      graph_features, fcW[:H2], fcW[H2:], fcb.reshape(1, NCLS))

    return out


# speedup vs baseline: 34.8936x; 1.1232x over previous
import functools

import jax
import jax.numpy as jnp
from jax import lax
from jax.experimental import pallas as pl
from jax.experimental.pallas import tpu as pltpu
from jax.experimental.pallas import tpu_sc as plsc

N = 10000
E = 320000
D_FEAT = 128
H1 = 16
H2 = 32
G = 64
GF = 16
NCLS = 10

NC = 2
NS = 16
NW = NC * NS
CH = 128
EPW = 10240
KCH = EPW // CH
EPAD = NW * EPW
NPAD = 10240
RPT = NPAD // NS
NB = 8

def _mesh():
    return plsc.VectorSubcoreMesh(
        core_axis_name="c", subcore_axis_name="s",
        num_cores=NC, num_subcores=NS)


@functools.cache
def _make_deg_kernel():
    @functools.partial(
        pl.kernel,
        out_type=jax.ShapeDtypeStruct((NC, NPAD), jnp.float32),
        mesh=_mesh(),
        compiler_params=pltpu.CompilerParams(use_tc_tiling_on_sc=False),
        scratch_types=[
            pltpu.VMEM((KCH, CH), jnp.int32),
            pltpu.VMEM((CH,), jnp.float32),
            pltpu.VMEM((RPT,), jnp.float32),
            pltpu.VMEM_SHARED((NPAD,), jnp.float32),
        ])
    def _deg_kernel(dst_hbm, out_hbm, dst_v, ones_v, zbuf, acc_sh):
        cid = lax.axis_index("c")
        sid = lax.axis_index("s")
        wid = sid * NC + cid

        one16 = jnp.ones((16,), jnp.float32)
        zero16 = jnp.zeros((16,), jnp.float32)

        def fill(i, _):
            ones_v[pl.ds(i * 16, 16)] = one16
            return 0
        lax.fori_loop(0, CH // 16, fill, 0)

        def zfill(i, _):
            zbuf[pl.ds(i * 16, 16)] = zero16
            return 0
        lax.fori_loop(0, RPT // 16, zfill, 0)
        pltpu.sync_copy(zbuf, acc_sh.at[pl.ds(sid * RPT, RPT)])
        plsc.subcore_barrier()

        pltpu.sync_copy(dst_hbm.at[pl.ds(wid * KCH, KCH)], dst_v)

        def body(j, _):
            pltpu.sync_copy(ones_v, acc_sh.at[dst_v.at[j]], add=True)
            return 0
        lax.fori_loop(0, KCH, body, 0)
        plsc.subcore_barrier()

        pltpu.sync_copy(acc_sh.at[pl.ds(sid * RPT, RPT)],
                        out_hbm.at[cid].at[pl.ds(sid * RPT, RPT)])
    return _deg_kernel


@functools.cache
def _make_scatter_kernel(F):
    @functools.partial(
        pl.kernel,
        out_type=jax.ShapeDtypeStruct((NC, NPAD, F), jnp.float32),
        mesh=_mesh(),
        compiler_params=pltpu.CompilerParams(use_tc_tiling_on_sc=False),
        scratch_types=[
            pltpu.VMEM((KCH, CH), jnp.int32),
            pltpu.VMEM((KCH, CH), jnp.int32),
            pltpu.VMEM((2 * NB, CH, F), jnp.float32),
            pltpu.VMEM((RPT, F), jnp.float32),
            pltpu.VMEM_SHARED((NPAD, F), jnp.float32),
            (pltpu.VMEM_SHARED((N, F), jnp.float32) if F == H1
             else pltpu.VMEM((8,), jnp.float32)),
            pltpu.SemaphoreType.DMA,
            pltpu.SemaphoreType.DMA,
        ])
    def k(table_hbm, src_hbm, dst_hbm, out_hbm,
          src_v, dst_v, rows_v, zbuf, acc_sh, table_sh, sem_g, sem_s):
        cid = lax.axis_index("c")
        sid = lax.axis_index("s")
        wid = sid * NC + cid

        zero16 = jnp.zeros((16,), jnp.float32)

        def zfill(i, _):
            for j in range(F // 16):
                zbuf[i, pl.ds(j * 16, 16)] = zero16
            return 0
        lax.fori_loop(0, RPT, zfill, 0)
        pltpu.sync_copy(zbuf, acc_sh.at[pl.ds(sid * RPT, RPT)])
        if F == H1:
            TPT = N // NS
            pltpu.sync_copy(table_hbm.at[pl.ds(sid * TPT, TPT)],
                            table_sh.at[pl.ds(sid * TPT, TPT)])
        tbl = table_sh if F == H1 else table_hbm
        plsc.subcore_barrier()

        pltpu.sync_copy(src_hbm.at[pl.ds(wid * KCH, KCH)], src_v)
        pltpu.sync_copy(dst_hbm.at[pl.ds(wid * KCH, KCH)], dst_v)

        NG = KCH // NB

        for i in range(NB):
            pltpu.async_copy(tbl.at[src_v.at[i]], rows_v.at[i], sem_g)

        def body(g, _):
            p = (g % 2) * NB
            q = ((g + 1) % 2) * NB
            for i in range(NB):
                pltpu.make_async_copy(
                    tbl.at[src_v.at[i]], rows_v.at[p + i], sem_g).wait()
            @pl.when(g > 0)
            def _():
                for i in range(NB):
                    pltpu.make_async_copy(
                        rows_v.at[q + i], acc_sh.at[dst_v.at[i]], sem_s).wait()
            @pl.when(g + 1 < NG)
            def _():
                for i in range(NB):
                    pltpu.async_copy(
                        tbl.at[src_v.at[(g + 1) * NB + i]],
                        rows_v.at[q + i], sem_g)
            for i in range(NB):
                pltpu.async_copy(rows_v.at[p + i],
                                 acc_sh.at[dst_v.at[g * NB + i]], sem_s,
                                 add=True)
            return 0
        lax.fori_loop(0, NG, body, 0)
        for i in range(NB):
            pltpu.make_async_copy(
                rows_v.at[i], acc_sh.at[dst_v.at[i]], sem_s).wait()
        plsc.subcore_barrier()

        pltpu.sync_copy(acc_sh.at[pl.ds(sid * RPT, RPT)],
                        out_hbm.at[cid].at[pl.ds(sid * RPT, RPT)])
    return k


def _k1_body(x_ref, w1_ref, degp_ref, dinv_ref, hs1_ref):
    degp = degp_ref[...]
    deg = degp[:, 0:1] + degp[:, 1:2] + 1.0
    dinv = lax.rsqrt(deg)
    dinv_ref[...] = dinv
    h = jnp.dot(x_ref[...], w1_ref[...], preferred_element_type=jnp.float32)
    hs1_ref[...] = h * dinv[:N]


def _k3_body(accp_ref, hs1_ref, dinv_ref, b1_ref, w2_ref, hs2_ref):
    acc = accp_ref[0, :N, :] + accp_ref[1, :N, :]
    dinv = dinv_ref[...][:N]
    t = jax.nn.relu(dinv * (acc + hs1_ref[...]) + b1_ref[...])
    hs2_ref[...] = jnp.dot(t, w2_ref[...],
                           preferred_element_type=jnp.float32) * dinv


def _k5_body(accp_ref, hs2_ref, dinv_ref, b2_ref, batch_ref, gf_ref,
             fcWh_ref, fcWg_ref, fcb_ref, out_ref):
    acc = accp_ref[0, :N, :] + accp_ref[1, :N, :]
    dinv = dinv_ref[...][:N]
    h2 = jax.nn.relu(dinv * (acc + hs2_ref[...]) + b2_ref[...])
    gid = lax.broadcasted_iota(jnp.int32, (G, N), 0)
    onehot = (gid == batch_ref[...]).astype(jnp.float32)
    counts = jnp.sum(onehot, axis=1, keepdims=True)
    sums = jnp.dot(onehot, h2, preferred_element_type=jnp.float32)
    pooled = sums / jnp.maximum(counts, 1.0)
    z = (jnp.dot(pooled, fcWh_ref[...], preferred_element_type=jnp.float32)
         + jnp.dot(gf_ref[...], fcWg_ref[...],
                   preferred_element_type=jnp.float32)
         + fcb_ref[...])
    m = jnp.max(z, axis=1, keepdims=True)
    lse = m + jnp.log(jnp.sum(jnp.exp(z - m), axis=1, keepdims=True))
    out_ref[...] = z - lse


def kernel(x, edge_index, batch, graph_features, W1, b1, W2, b2, fcW, fcb):
    src = edge_index[0].astype(jnp.int32)
    dst = edge_index[1].astype(jnp.int32)
    npad = EPAD - E
    srcp = jnp.concatenate([src, jnp.zeros((npad,), jnp.int32)])
    dstp = jnp.concatenate([dst, jnp.full((npad,), N, jnp.int32)])
    srcm = srcp.reshape(NW * KCH, CH)
    dstm = dstp.reshape(NW * KCH, CH)

    degp = _make_deg_kernel()(dstm)
    degp_t = degp.T

    dinv, hs1 = pl.pallas_call(
        _k1_body,
        out_shape=[jax.ShapeDtypeStruct((NPAD, 1), jnp.float32),
                   jax.ShapeDtypeStruct((N, H1), jnp.float32)],
    )(x, W1, degp_t)

    acc1 = _make_scatter_kernel(H1)(hs1, srcm, dstm)

    hs2 = pl.pallas_call(
        _k3_body,
        out_shape=jax.ShapeDtypeStruct((N, H2), jnp.float32),
    )(acc1, hs1, dinv, b1.reshape(1, H1), W2)

    acc2 = _make_scatter_kernel(H2)(hs2, srcm, dstm)

    out = pl.pallas_call(
        _k5_body,
        out_shape=jax.ShapeDtypeStruct((G, NCLS), jnp.float32),
    )(acc2, hs2, dinv, b2.reshape(1, H2), batch.astype(jnp.int32).reshape(1, N),
      graph_features, fcW[:H2], fcW[H2:], fcb.reshape(1, NCLS))

    return out


# speedup vs baseline: 47.3041x; 1.3557x over previous
import functools

import jax
import jax.numpy as jnp
from jax import lax
from jax.experimental import pallas as pl
from jax.experimental.pallas import tpu as pltpu
from jax.experimental.pallas import tpu_sc as plsc

N = 10000
E = 320000
D_FEAT = 128
H1 = 16
H2 = 32
G = 64
GF = 16
NCLS = 10

NC = 2
NS = 16
NW = NC * NS
CH = 128
EPW = 10240
KCH = EPW // CH
EPAD = NW * EPW
NPAD = 10240
RPT = NPAD // NS
NB = 8

def _mesh():
    return plsc.VectorSubcoreMesh(
        core_axis_name="c", subcore_axis_name="s",
        num_cores=NC, num_subcores=NS)


@functools.cache
def _make_deg_kernel():
    @functools.partial(
        pl.kernel,
        out_type=jax.ShapeDtypeStruct((NC, NPAD), jnp.float32),
        mesh=_mesh(),
        compiler_params=pltpu.CompilerParams(use_tc_tiling_on_sc=False),
        scratch_types=[
            pltpu.VMEM((KCH, CH), jnp.int32),
            pltpu.VMEM((CH,), jnp.float32),
            pltpu.VMEM((RPT,), jnp.float32),
            pltpu.VMEM_SHARED((NPAD,), jnp.float32),
        ])
    def _deg_kernel(dst_hbm, out_hbm, dst_v, ones_v, zbuf, acc_sh):
        cid = lax.axis_index("c")
        sid = lax.axis_index("s")
        wid = sid * NC + cid

        one16 = jnp.ones((16,), jnp.float32)
        zero16 = jnp.zeros((16,), jnp.float32)

        def fill(i, _):
            ones_v[pl.ds(i * 16, 16)] = one16
            return 0
        lax.fori_loop(0, CH // 16, fill, 0)

        def zfill(i, _):
            zbuf[pl.ds(i * 16, 16)] = zero16
            return 0
        lax.fori_loop(0, RPT // 16, zfill, 0)
        pltpu.sync_copy(zbuf, acc_sh.at[pl.ds(sid * RPT, RPT)])
        plsc.subcore_barrier()

        pltpu.sync_copy(dst_hbm.at[pl.ds(wid * KCH, KCH)], dst_v)

        def body(j, _):
            pltpu.sync_copy(ones_v, acc_sh.at[dst_v.at[j]], add=True)
            return 0
        lax.fori_loop(0, KCH, body, 0)
        plsc.subcore_barrier()

        pltpu.sync_copy(acc_sh.at[pl.ds(sid * RPT, RPT)],
                        out_hbm.at[cid].at[pl.ds(sid * RPT, RPT)])
    return _deg_kernel


@functools.cache
def _make_scatter_kernel(F):
    @functools.partial(
        pl.kernel,
        out_type=jax.ShapeDtypeStruct((NC, NPAD, F), jnp.float32),
        mesh=_mesh(),
        compiler_params=pltpu.CompilerParams(use_tc_tiling_on_sc=False),
        scratch_types=[
            pltpu.VMEM((KCH, CH), jnp.int32),
            pltpu.VMEM((KCH, CH), jnp.int32),
            pltpu.VMEM((2 * NB, CH, F), jnp.float32),
            pltpu.VMEM((RPT, F), jnp.float32),
            pltpu.VMEM_SHARED((NPAD, F), jnp.float32),
            (pltpu.VMEM_SHARED((N, F), jnp.float32) if F == H1
             else pltpu.VMEM((8,), jnp.float32)),
            pltpu.SemaphoreType.DMA,
            pltpu.SemaphoreType.DMA,
        ])
    def k(table_hbm, src_hbm, dst_hbm, out_hbm,
          src_v, dst_v, rows_v, zbuf, acc_sh, table_sh, sem_g, sem_s):
        cid = lax.axis_index("c")
        sid = lax.axis_index("s")
        wid = sid * NC + cid

        zero16 = jnp.zeros((16,), jnp.float32)

        def zfill(i, _):
            for j in range(F // 16):
                zbuf[i, pl.ds(j * 16, 16)] = zero16
            return 0
        lax.fori_loop(0, RPT, zfill, 0)
        pltpu.sync_copy(zbuf, acc_sh.at[pl.ds(sid * RPT, RPT)])
        if F == H1:
            TPT = N // NS
            pltpu.sync_copy(table_hbm.at[pl.ds(sid * TPT, TPT)],
                            table_sh.at[pl.ds(sid * TPT, TPT)])
        tbl = table_sh if F == H1 else table_hbm
        plsc.subcore_barrier()

        pltpu.sync_copy(src_hbm.at[pl.ds(wid * KCH, KCH)], src_v)
        pltpu.sync_copy(dst_hbm.at[pl.ds(wid * KCH, KCH)], dst_v)

        NG = KCH // NB

        for i in range(NB):
            pltpu.async_copy(tbl.at[src_v.at[i]], rows_v.at[i], sem_g)

        def body(g, _):
            p = (g % 2) * NB
            q = ((g + 1) % 2) * NB
            for i in range(NB):
                pltpu.make_async_copy(
                    tbl.at[src_v.at[i]], rows_v.at[p + i], sem_g).wait()
            @pl.when(g > 0)
            def _():
                for i in range(NB):
                    pltpu.make_async_copy(
                        rows_v.at[q + i], acc_sh.at[dst_v.at[i]], sem_s).wait()
            @pl.when(g + 1 < NG)
            def _():
                for i in range(NB):
                    pltpu.async_copy(
                        tbl.at[src_v.at[(g + 1) * NB + i]],
                        rows_v.at[q + i], sem_g)
            for i in range(NB):
                pltpu.async_copy(rows_v.at[p + i],
                                 acc_sh.at[dst_v.at[g * NB + i]], sem_s,
                                 add=True)
            return 0
        lax.fori_loop(0, NG, body, 0)
        for i in range(NB):
            pltpu.make_async_copy(
                rows_v.at[i], acc_sh.at[dst_v.at[i]], sem_s).wait()
        plsc.subcore_barrier()

        pltpu.sync_copy(acc_sh.at[pl.ds(sid * RPT, RPT)],
                        out_hbm.at[cid].at[pl.ds(sid * RPT, RPT)])
    return k


def _k1_body(x_ref, w1_ref, degp_ref, dinv_ref, hs1_ref):
    degp = degp_ref[...]
    deg = degp[:, 0:1] + degp[:, 1:2] + 1.0
    dinv = lax.rsqrt(deg)
    dinv_ref[...] = dinv
    h = jnp.dot(x_ref[...], w1_ref[...], preferred_element_type=jnp.float32)
    hs1_ref[...] = h * dinv[:N]


def _k3_body(accp_ref, hs1_ref, dinv_ref, b1_ref, w2a_ref, w2b_ref,
             hs2a_ref, hs2b_ref):
    acc = accp_ref[0, :N, :] + accp_ref[1, :N, :]
    dinv = dinv_ref[...][:N]
    t = jax.nn.relu(dinv * (acc + hs1_ref[...]) + b1_ref[...])
    hs2a_ref[...] = jnp.dot(t, w2a_ref[...],
                            preferred_element_type=jnp.float32) * dinv
    hs2b_ref[...] = jnp.dot(t, w2b_ref[...],
                            preferred_element_type=jnp.float32) * dinv


def _k5_body(accpa_ref, accpb_ref, hs2a_ref, hs2b_ref, dinv_ref, b2_ref,
             batch_ref, gf_ref, fcWh_ref, fcWg_ref, fcb_ref, out_ref):
    acca = accpa_ref[0, :N, :] + accpa_ref[1, :N, :]
    accb = accpb_ref[0, :N, :] + accpb_ref[1, :N, :]
    acc = jnp.concatenate([acca, accb], axis=1)
    hs2 = jnp.concatenate([hs2a_ref[...], hs2b_ref[...]], axis=1)
    dinv = dinv_ref[...][:N]
    h2 = jax.nn.relu(dinv * (acc + hs2) + b2_ref[...])
    gid = lax.broadcasted_iota(jnp.int32, (G, N), 0)
    onehot = (gid == batch_ref[...]).astype(jnp.float32)
    counts = jnp.sum(onehot, axis=1, keepdims=True)
    sums = jnp.dot(onehot, h2, preferred_element_type=jnp.float32)
    pooled = sums / jnp.maximum(counts, 1.0)
    z = (jnp.dot(pooled, fcWh_ref[...], preferred_element_type=jnp.float32)
         + jnp.dot(gf_ref[...], fcWg_ref[...],
                   preferred_element_type=jnp.float32)
         + fcb_ref[...])
    m = jnp.max(z, axis=1, keepdims=True)
    lse = m + jnp.log(jnp.sum(jnp.exp(z - m), axis=1, keepdims=True))
    out_ref[...] = z - lse


def kernel(x, edge_index, batch, graph_features, W1, b1, W2, b2, fcW, fcb):
    src = edge_index[0].astype(jnp.int32)
    dst = edge_index[1].astype(jnp.int32)
    npad = EPAD - E
    srcp = jnp.concatenate([src, jnp.zeros((npad,), jnp.int32)])
    dstp = jnp.concatenate([dst, jnp.full((npad,), N, jnp.int32)])
    srcm = srcp.reshape(NW * KCH, CH)
    dstm = dstp.reshape(NW * KCH, CH)

    degp = _make_deg_kernel()(dstm)
    degp_t = degp.T

    dinv, hs1 = pl.pallas_call(
        _k1_body,
        out_shape=[jax.ShapeDtypeStruct((NPAD, 1), jnp.float32),
                   jax.ShapeDtypeStruct((N, H1), jnp.float32)],
    )(x, W1, degp_t)

    acc1 = _make_scatter_kernel(H1)(hs1, srcm, dstm)

    hs2a, hs2b = pl.pallas_call(
        _k3_body,
        out_shape=[jax.ShapeDtypeStruct((N, H1), jnp.float32),
                   jax.ShapeDtypeStruct((N, H1), jnp.float32)],
    )(acc1, hs1, dinv, b1.reshape(1, H1), W2[:, :H1], W2[:, H1:])

    acc2a = _make_scatter_kernel(H1)(hs2a, srcm, dstm)
    acc2b = _make_scatter_kernel(H1)(hs2b, srcm, dstm)

    out = pl.pallas_call(
        _k5_body,
        out_shape=jax.ShapeDtypeStruct((G, NCLS), jnp.float32),
    )(acc2a, acc2b, hs2a, hs2b, dinv, b2.reshape(1, H2),
      batch.astype(jnp.int32).reshape(1, N),
      graph_features, fcW[:H2], fcW[H2:], fcb.reshape(1, NCLS))

    return out


# speedup vs baseline: 49.9338x; 1.0556x over previous
import functools

import jax
import jax.numpy as jnp
from jax import lax
from jax.experimental import pallas as pl
from jax.experimental.pallas import tpu as pltpu
from jax.experimental.pallas import tpu_sc as plsc

N = 10000
E = 320000
D_FEAT = 128
H1 = 16
H2 = 32
G = 64
GF = 16
NCLS = 10

NC = 2
NS = 16
NW = NC * NS
CH = 125
KCH = 80
NPAD = 10240
RPT = NPAD // NS
NB = 8

def _mesh():
    return plsc.VectorSubcoreMesh(
        core_axis_name="c", subcore_axis_name="s",
        num_cores=NC, num_subcores=NS)


@functools.cache
def _make_deg_kernel():
    @functools.partial(
        pl.kernel,
        out_type=jax.ShapeDtypeStruct((NC * NPAD,), jnp.float32),
        mesh=_mesh(),
        compiler_params=pltpu.CompilerParams(use_tc_tiling_on_sc=False),
        scratch_types=[
            pltpu.VMEM((KCH, CH), jnp.int32),
            pltpu.VMEM((128,), jnp.float32),
            pltpu.VMEM((RPT,), jnp.float32),
            pltpu.VMEM_SHARED((NPAD,), jnp.float32),
        ])
    def _deg_kernel(dst_hbm, out_hbm, dst_v, ones_v, zbuf, acc_sh):
        cid = lax.axis_index("c")
        sid = lax.axis_index("s")
        wid = sid * NC + cid

        one16 = jnp.ones((16,), jnp.float32)
        zero16 = jnp.zeros((16,), jnp.float32)

        def fill(i, _):
            ones_v[pl.ds(i * 16, 16)] = one16
            return 0
        lax.fori_loop(0, 128 // 16, fill, 0)

        def zfill(i, _):
            zbuf[pl.ds(i * 16, 16)] = zero16
            return 0
        lax.fori_loop(0, RPT // 16, zfill, 0)
        pltpu.sync_copy(zbuf, acc_sh.at[pl.ds(sid * RPT, RPT)])
        plsc.subcore_barrier()

        pltpu.sync_copy(dst_hbm.at[pl.ds(wid * KCH, KCH)], dst_v)

        def body(j, _):
            pltpu.sync_copy(ones_v.at[pl.ds(0, CH)],
                            acc_sh.at[dst_v.at[j]], add=True)
            return 0
        lax.fori_loop(0, KCH, body, 0)
        plsc.subcore_barrier()

        pltpu.sync_copy(acc_sh.at[pl.ds(sid * RPT, RPT)],
                        out_hbm.at[pl.ds(cid * NPAD + sid * RPT, RPT)])
    return _deg_kernel


@functools.cache
def _make_scatter_kernel(nhalves):
    F = H1
    @functools.partial(
        pl.kernel,
        out_type=[jax.ShapeDtypeStruct((NC * NPAD, F), jnp.float32)
                  for _ in range(nhalves)],
        mesh=_mesh(),
        compiler_params=pltpu.CompilerParams(use_tc_tiling_on_sc=False),
        scratch_types=(
            [pltpu.VMEM((KCH, CH), jnp.int32)] * 2 +
            [pltpu.VMEM((2 * NB, CH, F), jnp.float32)] * nhalves +
            [pltpu.VMEM((RPT, F), jnp.float32)] +
            [pltpu.VMEM_SHARED((NPAD, F), jnp.float32)] * nhalves +
            [pltpu.VMEM_SHARED((N, F), jnp.float32)] * nhalves +
            [pltpu.SemaphoreType.DMA] * 2
        ))
    def k(*args):
        tables_hbm = args[0:nhalves]
        src_hbm, dst_hbm = args[nhalves], args[nhalves + 1]
        outs_hbm = args[nhalves + 2:2 * nhalves + 2]
        a = 2 * nhalves + 2
        src_v, dst_v = args[a], args[a + 1]
        rings = args[a + 2:a + 2 + nhalves]
        zbuf = args[a + 2 + nhalves]
        accs_sh = args[a + 3 + nhalves:a + 3 + 2 * nhalves]
        tables_sh = args[a + 3 + 2 * nhalves:a + 3 + 3 * nhalves]
        sem_g, sem_s = args[a + 3 + 3 * nhalves], args[a + 4 + 3 * nhalves]

        cid = lax.axis_index("c")
        sid = lax.axis_index("s")
        wid = sid * NC + cid

        zero16 = jnp.zeros((16,), jnp.float32)

        def zfill(i, _):
            zbuf[i, pl.ds(0, 16)] = zero16
            return 0
        lax.fori_loop(0, RPT, zfill, 0)
        for h in range(nhalves):
            pltpu.sync_copy(zbuf, accs_sh[h].at[pl.ds(sid * RPT, RPT)])
        TPT = N // NS
        for h in range(nhalves):
            pltpu.sync_copy(tables_hbm[h].at[pl.ds(sid * TPT, TPT)],
                            tables_sh[h].at[pl.ds(sid * TPT, TPT)])
        plsc.subcore_barrier()

        pltpu.sync_copy(src_hbm.at[pl.ds(wid * KCH, KCH)], src_v)
        pltpu.sync_copy(dst_hbm.at[pl.ds(wid * KCH, KCH)], dst_v)

        NG = KCH // NB

        for i in range(NB):
            for h in range(nhalves):
                pltpu.async_copy(tables_sh[h].at[src_v.at[i]],
                                 rings[h].at[i], sem_g)

        def body(g, _):
            p = (g % 2) * NB
            q = ((g + 1) % 2) * NB
            for i in range(NB):
                for h in range(nhalves):
                    pltpu.make_async_copy(
                        tables_sh[h].at[src_v.at[i]],
                        rings[h].at[p + i], sem_g).wait()
            @pl.when(g > 0)
            def _():
                for i in range(NB):
                    for h in range(nhalves):
                        pltpu.make_async_copy(
                            rings[h].at[q + i],
                            accs_sh[h].at[dst_v.at[i]], sem_s).wait()
            @pl.when(g + 1 < NG)
            def _():
                for i in range(NB):
                    for h in range(nhalves):
                        pltpu.async_copy(
                            tables_sh[h].at[src_v.at[(g + 1) * NB + i]],
                            rings[h].at[q + i], sem_g)
            for i in range(NB):
                for h in range(nhalves):
                    pltpu.async_copy(rings[h].at[p + i],
                                     accs_sh[h].at[dst_v.at[g * NB + i]],
                                     sem_s, add=True)
            return 0
        lax.fori_loop(0, NG, body, 0)
        for i in range(NB):
            for h in range(nhalves):
                pltpu.make_async_copy(
                    rings[h].at[i], accs_sh[h].at[dst_v.at[i]], sem_s).wait()
        plsc.subcore_barrier()

        for h in range(nhalves):
            pltpu.sync_copy(
                accs_sh[h].at[pl.ds(sid * RPT, RPT)],
                outs_hbm[h].at[pl.ds(cid * NPAD + sid * RPT, RPT)])
    return k


def _k1a_body(x_ref, w1_ref, batch_ref, h1_ref, onehot_ref, counts_ref):
    h1_ref[...] = jnp.dot(x_ref[...], w1_ref[...],
                          preferred_element_type=jnp.float32)
    gid = lax.broadcasted_iota(jnp.int32, (G, N), 0)
    onehot = (gid == batch_ref[...]).astype(jnp.float32)
    onehot_ref[...] = onehot
    counts_ref[...] = jnp.sum(onehot, axis=1, keepdims=True)


def _k1b_body(degp_ref, h1_ref, dinv_ref, hs1_ref):
    deg = degp_ref[0:NPAD] + degp_ref[NPAD:2 * NPAD] + 1.0
    dinv = lax.rsqrt(deg)
    dinv_ref[...] = dinv
    hs1_ref[...] = h1_ref[...] * dinv[:N]


def _k3_body(accp_ref, hs1_ref, dinv_ref, b1_ref, w2a_ref, w2b_ref,
             hs2a_ref, hs2b_ref):
    acc = accp_ref[0:N, :] + accp_ref[NPAD:NPAD + N, :]
    dinv = dinv_ref[...][:N]
    t = jax.nn.relu(dinv * (acc + hs1_ref[...]) + b1_ref[...])
    hs2a_ref[...] = jnp.dot(t, w2a_ref[...],
                            preferred_element_type=jnp.float32) * dinv
    hs2b_ref[...] = jnp.dot(t, w2b_ref[...],
                            preferred_element_type=jnp.float32) * dinv


def _k5_body(accpa_ref, accpb_ref, hs2a_ref, hs2b_ref, dinv_ref, b2_ref,
             onehot_ref, counts_ref, gf_ref, fcWh_ref, fcWg_ref, fcb_ref,
             out_ref):
    acca = accpa_ref[0:N, :] + accpa_ref[NPAD:NPAD + N, :]
    accb = accpb_ref[0:N, :] + accpb_ref[NPAD:NPAD + N, :]
    acc = jnp.concatenate([acca, accb], axis=1)
    hs2 = jnp.concatenate([hs2a_ref[...], hs2b_ref[...]], axis=1)
    dinv = dinv_ref[...][:N]
    h2 = jax.nn.relu(dinv * (acc + hs2) + b2_ref[...])
    sums = jnp.dot(onehot_ref[...], h2, preferred_element_type=jnp.float32)
    pooled = sums / jnp.maximum(counts_ref[...], 1.0)
    z = (jnp.dot(pooled, fcWh_ref[...], preferred_element_type=jnp.float32)
         + jnp.dot(gf_ref[...], fcWg_ref[...],
                   preferred_element_type=jnp.float32)
         + fcb_ref[...])
    m = jnp.max(z, axis=1, keepdims=True)
    lse = m + jnp.log(jnp.sum(jnp.exp(z - m), axis=1, keepdims=True))
    out_ref[...] = z - lse


def kernel(x, edge_index, batch, graph_features, W1, b1, W2, b2, fcW, fcb):
    srcm = edge_index[0].astype(jnp.int32).reshape(NW * KCH, CH)
    dstm = edge_index[1].astype(jnp.int32).reshape(NW * KCH, CH)

    h1, onehot, counts = pl.pallas_call(
        _k1a_body,
        out_shape=[jax.ShapeDtypeStruct((N, H1), jnp.float32),
                   jax.ShapeDtypeStruct((G, N), jnp.float32),
                   jax.ShapeDtypeStruct((G, 1), jnp.float32)],
    )(x, W1, batch.astype(jnp.int32).reshape(1, N))

    degp = _make_deg_kernel()(dstm)

    dinv, hs1 = pl.pallas_call(
        _k1b_body,
        out_shape=[jax.ShapeDtypeStruct((NPAD, 1), jnp.float32),
                   jax.ShapeDtypeStruct((N, H1), jnp.float32)],
    )(degp.reshape(2 * NPAD, 1), h1)

    (acc1,) = _make_scatter_kernel(1)(hs1, srcm, dstm)

    hs2a, hs2b = pl.pallas_call(
        _k3_body,
        out_shape=[jax.ShapeDtypeStruct((N, H1), jnp.float32),
                   jax.ShapeDtypeStruct((N, H1), jnp.float32)],
    )(acc1, hs1, dinv, b1.reshape(1, H1), W2[:, :H1], W2[:, H1:])

    (acc2a,) = _make_scatter_kernel(1)(hs2a, srcm, dstm)
    (acc2b,) = _make_scatter_kernel(1)(hs2b, srcm, dstm)

    out = pl.pallas_call(
        _k5_body,
        out_shape=jax.ShapeDtypeStruct((G, NCLS), jnp.float32),
    )(acc2a, acc2b, hs2a, hs2b, dinv, b2.reshape(1, H2), onehot, counts,
      graph_features, fcW[:H2], fcW[H2:], fcb.reshape(1, NCLS))

    return out


# speedup vs baseline: 52.3556x; 1.0485x over previous
import functools

import jax
import jax.numpy as jnp
from jax import lax
from jax.experimental import pallas as pl
from jax.experimental.pallas import tpu as pltpu
from jax.experimental.pallas import tpu_sc as plsc

N = 10000
E = 320000
D_FEAT = 128
H1 = 16
H2 = 32
G = 64
GF = 16
NCLS = 10

NC = 2
NS = 16
NW = NC * NS
CH = 125
KCH = 80
NPAD = 10240
RPT = NPAD // NS
NB = 8

def _mesh():
    return plsc.VectorSubcoreMesh(
        core_axis_name="c", subcore_axis_name="s",
        num_cores=NC, num_subcores=NS)


@functools.cache
def _make_deg_kernel():
    @functools.partial(
        pl.kernel,
        out_type=jax.ShapeDtypeStruct((NC * NPAD,), jnp.float32),
        mesh=_mesh(),
        compiler_params=pltpu.CompilerParams(use_tc_tiling_on_sc=False),
        scratch_types=[
            pltpu.VMEM((KCH, CH), jnp.int32),
            pltpu.VMEM((128,), jnp.float32),
            pltpu.VMEM((RPT,), jnp.float32),
            pltpu.VMEM_SHARED((NPAD,), jnp.float32),
        ])
    def _deg_kernel(em_hbm, out_hbm, dst_v, ones_v, zbuf, acc_sh):
        cid = lax.axis_index("c")
        sid = lax.axis_index("s")
        wid = sid * NC + cid

        one16 = jnp.ones((16,), jnp.float32)
        zero16 = jnp.zeros((16,), jnp.float32)

        def fill(i, _):
            ones_v[pl.ds(i * 16, 16)] = one16
            return 0
        lax.fori_loop(0, 128 // 16, fill, 0)

        def zfill(i, _):
            zbuf[pl.ds(i * 16, 16)] = zero16
            return 0
        lax.fori_loop(0, RPT // 16, zfill, 0)
        pltpu.sync_copy(zbuf, acc_sh.at[pl.ds(sid * RPT, RPT)])
        plsc.subcore_barrier()

        pltpu.sync_copy(em_hbm.at[1].at[pl.ds(wid * KCH, KCH)], dst_v)

        def body(j, _):
            pltpu.sync_copy(ones_v.at[pl.ds(0, CH)],
                            acc_sh.at[dst_v.at[j]], add=True)
            return 0
        lax.fori_loop(0, KCH, body, 0)
        plsc.subcore_barrier()

        pltpu.sync_copy(acc_sh.at[pl.ds(sid * RPT, RPT)],
                        out_hbm.at[pl.ds(cid * NPAD + sid * RPT, RPT)])
    return _deg_kernel


@functools.cache
def _make_scatter_kernel(nhalves):
    F = H1
    @functools.partial(
        pl.kernel,
        out_type=[jax.ShapeDtypeStruct((NC * NPAD, F), jnp.float32)
                  for _ in range(nhalves)],
        mesh=_mesh(),
        compiler_params=pltpu.CompilerParams(use_tc_tiling_on_sc=False),
        scratch_types=(
            [pltpu.VMEM((KCH, CH), jnp.int32)] * 2 +
            [pltpu.VMEM((2 * NB, CH, F), jnp.float32)] * nhalves +
            [pltpu.VMEM((RPT, F), jnp.float32)] +
            [pltpu.VMEM_SHARED((NPAD, F), jnp.float32)] * nhalves +
            [pltpu.VMEM_SHARED((N, F), jnp.float32)] * nhalves +
            [pltpu.SemaphoreType.DMA] * 2
        ))
    def k(*args):
        tables_hbm = args[0:nhalves]
        em_hbm = args[nhalves]
        outs_hbm = args[nhalves + 1:2 * nhalves + 1]
        a = 2 * nhalves + 1
        src_v, dst_v = args[a], args[a + 1]
        rings = args[a + 2:a + 2 + nhalves]
        zbuf = args[a + 2 + nhalves]
        accs_sh = args[a + 3 + nhalves:a + 3 + 2 * nhalves]
        tables_sh = args[a + 3 + 2 * nhalves:a + 3 + 3 * nhalves]
        sem_g, sem_s = args[a + 3 + 3 * nhalves], args[a + 4 + 3 * nhalves]

        cid = lax.axis_index("c")
        sid = lax.axis_index("s")
        wid = sid * NC + cid

        zero16 = jnp.zeros((16,), jnp.float32)

        def zfill(i, _):
            zbuf[i, pl.ds(0, 16)] = zero16
            return 0
        lax.fori_loop(0, RPT, zfill, 0)
        for h in range(nhalves):
            pltpu.sync_copy(zbuf, accs_sh[h].at[pl.ds(sid * RPT, RPT)])
        TPT = N // NS
        for h in range(nhalves):
            pltpu.sync_copy(tables_hbm[h].at[pl.ds(sid * TPT, TPT)],
                            tables_sh[h].at[pl.ds(sid * TPT, TPT)])
        plsc.subcore_barrier()

        pltpu.sync_copy(em_hbm.at[0].at[pl.ds(wid * KCH, KCH)], src_v)
        pltpu.sync_copy(em_hbm.at[1].at[pl.ds(wid * KCH, KCH)], dst_v)

        NG = KCH // NB

        for i in range(NB):
            for h in range(nhalves):
                pltpu.async_copy(tables_sh[h].at[src_v.at[i]],
                                 rings[h].at[i], sem_g)

        def body(g, _):
            p = (g % 2) * NB
            q = ((g + 1) % 2) * NB
            for i in range(NB):
                for h in range(nhalves):
                    pltpu.make_async_copy(
                        tables_sh[h].at[src_v.at[i]],
                        rings[h].at[p + i], sem_g).wait()
            @pl.when(g > 0)
            def _():
                for i in range(NB):
                    for h in range(nhalves):
                        pltpu.make_async_copy(
                            rings[h].at[q + i],
                            accs_sh[h].at[dst_v.at[i]], sem_s).wait()
            @pl.when(g + 1 < NG)
            def _():
                for i in range(NB):
                    for h in range(nhalves):
                        pltpu.async_copy(
                            tables_sh[h].at[src_v.at[(g + 1) * NB + i]],
                            rings[h].at[q + i], sem_g)
            for i in range(NB):
                for h in range(nhalves):
                    pltpu.async_copy(rings[h].at[p + i],
                                     accs_sh[h].at[dst_v.at[g * NB + i]],
                                     sem_s, add=True)
            return 0
        lax.fori_loop(0, NG, body, 0)
        for i in range(NB):
            for h in range(nhalves):
                pltpu.make_async_copy(
                    rings[h].at[i], accs_sh[h].at[dst_v.at[i]], sem_s).wait()
        plsc.subcore_barrier()

        for h in range(nhalves):
            pltpu.sync_copy(
                accs_sh[h].at[pl.ds(sid * RPT, RPT)],
                outs_hbm[h].at[pl.ds(cid * NPAD + sid * RPT, RPT)])
    return k


def _k1a_body(x_ref, w1_ref, batch_ref, h1_ref, onehot_ref, counts_ref):
    h1_ref[...] = jnp.dot(x_ref[...], w1_ref[...],
                          preferred_element_type=jnp.float32)
    gid = lax.broadcasted_iota(jnp.int32, (G, N), 0)
    onehot = (gid == batch_ref[...]).astype(jnp.float32)
    onehot_ref[...] = onehot
    counts_ref[...] = jnp.sum(onehot, axis=1, keepdims=True)


def _k1b_body(degp_ref, h1_ref, dinv_ref, hs1_ref):
    deg = degp_ref[0:NPAD] + degp_ref[NPAD:2 * NPAD] + 1.0
    dinv = lax.rsqrt(deg)
    dinv_ref[...] = dinv
    hs1_ref[...] = h1_ref[...] * dinv[:N]


def _k3_body(accp_ref, hs1_ref, dinv_ref, b1_ref, w2a_ref, w2b_ref,
             hs2a_ref, hs2b_ref):
    acc = accp_ref[0:N, :] + accp_ref[NPAD:NPAD + N, :]
    dinv = dinv_ref[...][:N]
    t = jax.nn.relu(dinv * (acc + hs1_ref[...]) + b1_ref[...])
    hs2a_ref[...] = jnp.dot(t, w2a_ref[...],
                            preferred_element_type=jnp.float32) * dinv
    hs2b_ref[...] = jnp.dot(t, w2b_ref[...],
                            preferred_element_type=jnp.float32) * dinv


def _k5_body(accpa_ref, accpb_ref, hs2a_ref, hs2b_ref, dinv_ref, b2_ref,
             onehot_ref, counts_ref, gf_ref, fcWh_ref, fcWg_ref, fcb_ref,
             out_ref):
    acca = accpa_ref[0:N, :] + accpa_ref[NPAD:NPAD + N, :]
    accb = accpb_ref[0:N, :] + accpb_ref[NPAD:NPAD + N, :]
    acc = jnp.concatenate([acca, accb], axis=1)
    hs2 = jnp.concatenate([hs2a_ref[...], hs2b_ref[...]], axis=1)
    dinv = dinv_ref[...][:N]
    h2 = jax.nn.relu(dinv * (acc + hs2) + b2_ref[...])
    sums = jnp.dot(onehot_ref[...], h2, preferred_element_type=jnp.float32)
    pooled = sums / jnp.maximum(counts_ref[...], 1.0)
    z = (jnp.dot(pooled, fcWh_ref[...], preferred_element_type=jnp.float32)
         + jnp.dot(gf_ref[...], fcWg_ref[...],
                   preferred_element_type=jnp.float32)
         + fcb_ref[...])
    m = jnp.max(z, axis=1, keepdims=True)
    lse = m + jnp.log(jnp.sum(jnp.exp(z - m), axis=1, keepdims=True))
    out_ref[...] = z - lse


def kernel(x, edge_index, batch, graph_features, W1, b1, W2, b2, fcW, fcb):
    em = edge_index.astype(jnp.int32).reshape(2, NW * KCH, CH)

    h1, onehot, counts = pl.pallas_call(
        _k1a_body,
        out_shape=[jax.ShapeDtypeStruct((N, H1), jnp.float32),
                   jax.ShapeDtypeStruct((G, N), jnp.float32),
                   jax.ShapeDtypeStruct((G, 1), jnp.float32)],
    )(x, W1, batch.astype(jnp.int32).reshape(1, N))

    degp = _make_deg_kernel()(em)

    dinv, hs1 = pl.pallas_call(
        _k1b_body,
        out_shape=[jax.ShapeDtypeStruct((NPAD, 1), jnp.float32),
                   jax.ShapeDtypeStruct((N, H1), jnp.float32)],
    )(degp.reshape(2 * NPAD, 1), h1)

    (acc1,) = _make_scatter_kernel(1)(hs1, em)

    hs2a, hs2b = pl.pallas_call(
        _k3_body,
        out_shape=[jax.ShapeDtypeStruct((N, H1), jnp.float32),
                   jax.ShapeDtypeStruct((N, H1), jnp.float32)],
    )(acc1, hs1, dinv, b1.reshape(1, H1), W2[:, :H1], W2[:, H1:])

    (acc2a,) = _make_scatter_kernel(1)(hs2a, em)
    (acc2b,) = _make_scatter_kernel(1)(hs2b, em)

    out = pl.pallas_call(
        _k5_body,
        out_shape=jax.ShapeDtypeStruct((G, NCLS), jnp.float32),
    )(acc2a, acc2b, hs2a, hs2b, dinv, b2.reshape(1, H2), onehot, counts,
      graph_features, fcW[:H2], fcW[H2:], fcb.reshape(1, NCLS))

    return out
